# baseline (speedup 1.0000x reference)
import functools

import jax
import jax.numpy as jnp
from jax import lax
from jax.experimental import pallas as pl
from jax.experimental.pallas import tpu as pltpu
from jax.experimental.pallas import tpu_sc as plsc

NUM_CLASSES = 80
PRE_NMS_TOP_N = 1000
POST_NMS_TOP_N = 100
IOU_THR = 0.6
SCORE_THR = 0.05
CLASS_OFFSET = 4096.0

B = 2
K = 20000
KP = 20480
ROWS = 160
LANES = 128
BLK = 2000
BLKP = 2048
KB = K // BLK

NSUB = 16
CHUNK = KP // NSUB
NV = CHUNK // 16
SLOTS = 1024
SV = SLOTS // 16
SUBSLOTS = SLOTS // NSUB
CROWS = SLOTS // LANES


def _map_kernel(cls_ref, loc_ref, ctr_ref, locs_ref, score_ref, label_ref,
                box_ref):
  cls = cls_ref[0, 0]
  ctr = ctr_ref[0, 0]
  prob = jax.nn.sigmoid(cls) * jax.nn.sigmoid(ctr)
  m = jnp.max(prob, axis=1)
  lane = jax.lax.broadcasted_iota(jnp.int32, prob.shape, 1)
  amax = jnp.min(jnp.where(prob == m[:, None], lane, NUM_CLASSES), axis=1)
  zpad = jnp.zeros((BLKP - BLK,), jnp.float32)
  score_ref[0] = jnp.concatenate([m, zpad]).reshape(BLKP // LANES, LANES)
  label_ref[0] = jnp.concatenate([amax.astype(jnp.float32), zpad]).reshape(
      BLKP // LANES, LANES)
  off = loc_ref[0, 0]
  lx = locs_ref[0, :, 0]
  ly = locs_ref[0, :, 1]
  x1 = lx - off[:, 0]
  y1 = ly - off[:, 1]
  x2 = lx + off[:, 2]
  y2 = ly + off[:, 3]
  bx = jnp.stack([x1, y1, x2, y2], axis=0)
  box_ref[0] = jnp.pad(bx, ((0, 0), (0, BLKP - BLK))).reshape(
      4, BLKP // LANES, LANES)


def _select_kernel(score_ref, pos_ref):
  scores = score_ref[0]
  u = jax.lax.bitcast_convert_type(scores, jnp.int32)

  def bit_body(i, lo):
    t = lo + jnp.left_shift(jnp.int32(1), 30 - i)
    c = jnp.sum((u >= t).astype(jnp.int32))
    return jnp.where(c >= PRE_NMS_TOP_N, t, lo)

  thr = jax.lax.fori_loop(0, 31, bit_body, jnp.int32(0))
  gt = (u > thr)
  n_gt = jnp.sum(gt.astype(jnp.int32))
  need = PRE_NMS_TOP_N - n_gt
  tie = (u == thr)
  r_iota = jax.lax.broadcasted_iota(jnp.int32, (ROWS, ROWS), 0)
  c_iota = jax.lax.broadcasted_iota(jnp.int32, (ROWS, ROWS), 1)
  strict_l = (c_iota < r_iota).astype(jnp.float32)
  r2 = jax.lax.broadcasted_iota(jnp.int32, (LANES, LANES), 0)
  c2 = jax.lax.broadcasted_iota(jnp.int32, (LANES, LANES), 1)
  strict_u = (r2 < c2).astype(jnp.float32)

  def rank_of(mask):
    mf = mask.astype(jnp.float32)
    row_cnt = jnp.dot(mf, jnp.ones((LANES, 1), jnp.float32),
                      preferred_element_type=jnp.float32)
    row_pref = jnp.dot(strict_l, row_cnt,
                       preferred_element_type=jnp.float32)
    lane_pref = jnp.dot(mf, strict_u,
                        preferred_element_type=jnp.float32)
    return row_pref + lane_pref

  tie_rank = rank_of(tie)
  sel = gt | (tie & (tie_rank < need.astype(jnp.float32)))
  pos = rank_of(sel).astype(jnp.int32)
  pos_ref[0] = jnp.where(sel, pos, -1)


def _sc_compact_body(pos_hbm, score_hbm, label_hbm, box_hbm, out_hbm,
                     pos_v, buf_v, rows_v, comb_v, idx2_v, plane_v, gout_v,
                     sh1):
  cid = lax.axis_index("c")
  sid = lax.axis_index("s")
  base = pl.multiple_of(sid * CHUNK, 8)
  pltpu.sync_copy(pos_hbm.at[cid, pl.ds(base, CHUNK)], pos_v)

  def zero_body(i, carry):
    buf_v[pl.ds(pl.multiple_of(i * 16, 16), 16)] = jnp.zeros((16,), jnp.int32)
    return carry

  lax.fori_loop(0, SV, zero_body, 0)
  lane16 = lax.iota(jnp.int32, 16)

  def scat_body(j, carry):
    o = pl.multiple_of(j * 16, 16)
    p = pos_v[pl.ds(o, 16)]
    e = base + j * 16 + lane16
    plsc.store_scatter(buf_v, [p], e, mask=p >= 0)
    return carry

  lax.fori_loop(0, NV, scat_body, 0)
  pltpu.sync_copy(buf_v, sh1.at[sid])
  plsc.subcore_barrier()

  soff = pl.multiple_of(sid * SUBSLOTS, 8)
  for r in range(NSUB):
    pltpu.sync_copy(sh1.at[r, pl.ds(soff, SUBSLOTS)], rows_v.at[r])
  for v in range(SUBSLOTS // 16):
    acc = jnp.zeros((16,), jnp.int32)
    for r in range(NSUB):
      acc = acc + rows_v[r, pl.ds(v * 16, 16)]
    comb_v[pl.ds(v * 16, 16)] = acc
  for v in range(SUBSLOTS // 16, 128 // 16):
    comb_v[pl.ds(v * 16, 16)] = jnp.zeros((16,), jnp.int32)
  plsc.subcore_barrier()
  pltpu.sync_copy(comb_v, sh1.at[sid, pl.ds(0, 128)])
  plsc.subcore_barrier()

  @pl.when(sid < 6)
  def _gather():
    for r in range(NSUB):
      pltpu.sync_copy(sh1.at[r, pl.ds(0, SUBSLOTS)], idx2_v.at[r])
    for ch in range(4):
      @pl.when(sid == ch)
      def _load_box(ch=ch):
        pltpu.sync_copy(box_hbm.at[cid, ch], plane_v)
    @pl.when(sid == 4)
    def _load_score():
      pltpu.sync_copy(score_hbm.at[cid], plane_v)
    @pl.when(sid == 5)
    def _load_label():
      pltpu.sync_copy(label_hbm.at[cid], plane_v)
    for g in range(SV):
      iv = idx2_v[g // 4, pl.ds((g % 4) * 16, 16)]
      gout_v[pl.ds(g * 16, 16)] = plsc.load_gather(plane_v, [iv])
    pltpu.sync_copy(gout_v, out_hbm.at[cid, sid])


@functools.cache
def _sc_compact():
  return pl.kernel(
      _sc_compact_body,
      mesh=plsc.VectorSubcoreMesh(core_axis_name="c", subcore_axis_name="s"),
      compiler_params=pltpu.CompilerParams(needs_layout_passes=False),
      out_type=jax.ShapeDtypeStruct((B, 6, SLOTS), jnp.float32),
      scratch_types=[
          pltpu.VMEM((CHUNK,), jnp.int32),
          pltpu.VMEM((SLOTS,), jnp.int32),
          pltpu.VMEM((NSUB, SUBSLOTS), jnp.int32),
          pltpu.VMEM((128,), jnp.int32),
          pltpu.VMEM((NSUB, SUBSLOTS), jnp.int32),
          pltpu.VMEM((KP,), jnp.float32),
          pltpu.VMEM((SLOTS,), jnp.float32),
          pltpu.VMEM_SHARED((NSUB, SLOTS), jnp.int32),
      ],
  )


def _nms_kernel(cpt_ref, out_ref):
  x1 = cpt_ref[0, 0]
  y1 = cpt_ref[0, 1]
  x2 = cpt_ref[0, 2]
  y2 = cpt_ref[0, 3]
  scores = cpt_ref[0, 4]
  labels = cpt_ref[0, 5]
  lin = (jax.lax.broadcasted_iota(jnp.int32, (CROWS, LANES), 0) * LANES
         + jax.lax.broadcasted_iota(jnp.int32, (CROWS, LANES), 1))
  valid = lin < PRE_NMS_TOP_N
  s0 = jnp.where(valid & (scores > SCORE_THR), scores, -1e9)
  offs = labels * CLASS_OFFSET
  x1o = x1 + offs
  y1o = y1 + offs
  x2o = x2 + offs
  y2o = y2 + offs
  areas = jnp.maximum(x2o - x1o, 0.0) * jnp.maximum(y2o - y1o, 0.0)
  lane_i = jax.lax.broadcasted_iota(jnp.int32, (1, LANES), 1)

  def body(i, s):
    sc = jnp.max(s)
    il = jnp.min(jnp.where(s == sc, lin, jnp.int32(2**30)))
    r = il // LANES
    c = il % LANES

    def pick(row):
      return jnp.sum(jnp.where(lane_i == c, row, 0.0))

    bx1 = pick(cpt_ref[0, 0, pl.ds(r, 1), :])
    by1 = pick(cpt_ref[0, 1, pl.ds(r, 1), :])
    bx2 = pick(cpt_ref[0, 2, pl.ds(r, 1), :])
    by2 = pick(cpt_ref[0, 3, pl.ds(r, 1), :])
    lab = pick(cpt_ref[0, 5, pl.ds(r, 1), :])
    det = jnp.concatenate([
        jnp.full((1, 1), v, jnp.float32)
        for v in (bx1, by1, bx2, by2, sc, lab)
    ], axis=1)
    det = jnp.where(sc > -1e8, det, jnp.zeros_like(det))
    out_ref[0, pl.ds(i, 1), :] = det
    o = lab * CLASS_OFFSET
    b0 = bx1 + o
    b1 = by1 + o
    b2 = bx2 + o
    b3 = by2 + o
    xx1 = jnp.maximum(b0, x1o)
    yy1 = jnp.maximum(b1, y1o)
    xx2 = jnp.minimum(b2, x2o)
    yy2 = jnp.minimum(b3, y2o)
    inter = jnp.maximum(xx2 - xx1, 0.0) * jnp.maximum(yy2 - yy1, 0.0)
    a = jnp.maximum(b2 - b0, 0.0) * jnp.maximum(b3 - b1, 0.0)
    iou = inter / (a + areas - inter + 1e-9)
    s = jnp.where(iou > IOU_THR, -1e9, s)
    s = jnp.where(lin == il, -1e9, s)
    return s

  jax.lax.fori_loop(0, POST_NMS_TOP_N, body, s0)


@jax.jit
def kernel(cls_pred, loc_pred, ctr_pred, locations):
  cls4 = cls_pred.reshape(B, KB, BLK, NUM_CLASSES)
  loc4 = loc_pred.reshape(B, KB, BLK, 4)
  ctr4 = ctr_pred.reshape(B, KB, BLK, 1)
  locs3 = locations.reshape(KB, BLK, 2)

  scores, labels, boxes = pl.pallas_call(
      _map_kernel,
      grid=(B, KB),
      in_specs=[
          pl.BlockSpec((1, 1, BLK, NUM_CLASSES), lambda b, k: (b, k, 0, 0)),
          pl.BlockSpec((1, 1, BLK, 4), lambda b, k: (b, k, 0, 0)),
          pl.BlockSpec((1, 1, BLK, 1), lambda b, k: (b, k, 0, 0)),
          pl.BlockSpec((1, BLK, 2), lambda b, k: (k, 0, 0)),
      ],
      out_specs=[
          pl.BlockSpec((1, BLKP // LANES, LANES), lambda b, k: (b, k, 0)),
          pl.BlockSpec((1, BLKP // LANES, LANES), lambda b, k: (b, k, 0)),
          pl.BlockSpec((1, 4, BLKP // LANES, LANES),
                       lambda b, k: (b, 0, k, 0)),
      ],
      out_shape=[
          jax.ShapeDtypeStruct((B, ROWS, LANES), jnp.float32),
          jax.ShapeDtypeStruct((B, ROWS, LANES), jnp.float32),
          jax.ShapeDtypeStruct((B, 4, ROWS, LANES), jnp.float32),
      ],
  )(cls4, loc4, ctr4, locs3)

  pos = pl.pallas_call(
      _select_kernel,
      grid=(B,),
      in_specs=[pl.BlockSpec((1, ROWS, LANES), lambda b: (b, 0, 0))],
      out_specs=pl.BlockSpec((1, ROWS, LANES), lambda b: (b, 0, 0)),
      out_shape=jax.ShapeDtypeStruct((B, ROWS, LANES), jnp.int32),
  )(scores)

  compact = _sc_compact()(
      pos.reshape(B, KP), scores.reshape(B, KP), labels.reshape(B, KP),
      boxes.reshape(B, 4, KP))

  dets = pl.pallas_call(
      _nms_kernel,
      grid=(B,),
      in_specs=[pl.BlockSpec((1, 6, CROWS, LANES), lambda b: (b, 0, 0, 0))],
      out_specs=pl.BlockSpec((1, POST_NMS_TOP_N, 6), lambda b: (b, 0, 0)),
      out_shape=jax.ShapeDtypeStruct((B, POST_NMS_TOP_N, 6), jnp.float32),
  )(compact.reshape(B, 6, CROWS, LANES))
  return dets


# speedup vs baseline: 1.0874x; 1.0874x over previous
import functools

import jax
import jax.numpy as jnp
from jax import lax
from jax.experimental import pallas as pl
from jax.experimental.pallas import tpu as pltpu
from jax.experimental.pallas import tpu_sc as plsc

NUM_CLASSES = 80
PRE_NMS_TOP_N = 1000
POST_NMS_TOP_N = 100
IOU_THR = 0.6
SCORE_THR = 0.05
CLASS_OFFSET = 4096.0

B = 2
K = 20000
KP = 20480
ROWS = 160
LANES = 128
BLK = 2000
BLKP = 2048
KB = K // BLK

NSUB = 16
NSCAT = 10
CHUNK = KP // NSCAT
NV = CHUNK // 16
SLOTS = 1024
SV = SLOTS // 16
SUBSLOTS = SLOTS // NSUB
CROWS = SLOTS // LANES


def _map_kernel(cls_ref, loc_ref, ctr_ref, locs_ref, score_ref, label_ref,
                box_ref):
  cls = cls_ref[0]
  ctr = ctr_ref[0]
  prob = jax.nn.sigmoid(cls) * jax.nn.sigmoid(ctr)
  m = jnp.max(prob, axis=1)
  lane = jax.lax.broadcasted_iota(jnp.int32, prob.shape, 1)
  amax = jnp.min(jnp.where(prob == m[:, None], lane, NUM_CLASSES), axis=1)
  zpad = jnp.zeros((BLKP - BLK,), jnp.float32)
  score_ref[0] = jnp.concatenate([m, zpad]).reshape(BLKP // LANES, LANES)
  label_ref[0] = jnp.concatenate([amax.astype(jnp.float32), zpad]).reshape(
      BLKP // LANES, LANES)
  off = loc_ref[0]
  lx = locs_ref[:, 0]
  ly = locs_ref[:, 1]
  x1 = lx - off[:, 0]
  y1 = ly - off[:, 1]
  x2 = lx + off[:, 2]
  y2 = ly + off[:, 3]
  bx = jnp.stack([x1, y1, x2, y2], axis=0)
  box_ref[0] = jnp.pad(bx, ((0, 0), (0, BLKP - BLK))).reshape(
      4, BLKP // LANES, LANES)


def _select_kernel(score_ref, pos_ref):
  scores = score_ref[0]
  u = jax.lax.bitcast_convert_type(scores, jnp.int32)

  def bit_body(i, lo):
    t = lo + jnp.left_shift(jnp.int32(1), 30 - i)
    c = jnp.sum((u >= t).astype(jnp.int32))
    return jnp.where(c >= PRE_NMS_TOP_N, t, lo)

  thr = jax.lax.fori_loop(0, 31, bit_body, jnp.int32(0))
  gt = (u > thr)
  n_gt = jnp.sum(gt.astype(jnp.int32))
  need = PRE_NMS_TOP_N - n_gt
  tie = (u == thr)
  r_iota = jax.lax.broadcasted_iota(jnp.int32, (ROWS, ROWS), 0)
  c_iota = jax.lax.broadcasted_iota(jnp.int32, (ROWS, ROWS), 1)
  strict_l = (c_iota < r_iota).astype(jnp.float32)
  r2 = jax.lax.broadcasted_iota(jnp.int32, (LANES, LANES), 0)
  c2 = jax.lax.broadcasted_iota(jnp.int32, (LANES, LANES), 1)
  strict_u = (r2 < c2).astype(jnp.float32)

  def rank_of(mask):
    mf = mask.astype(jnp.float32)
    row_cnt = jnp.dot(mf, jnp.ones((LANES, 1), jnp.float32),
                      preferred_element_type=jnp.float32)
    row_pref = jnp.dot(strict_l, row_cnt,
                       preferred_element_type=jnp.float32)
    lane_pref = jnp.dot(mf, strict_u,
                        preferred_element_type=jnp.float32)
    return row_pref + lane_pref

  tie_rank = rank_of(tie)
  sel = gt | (tie & (tie_rank < need.astype(jnp.float32)))
  pos = rank_of(sel).astype(jnp.int32)
  pos_ref[0] = jnp.where(sel, pos, -1)


CROWS_PER_SUB = ROWS // NSCAT


def _sc_compact_body(pos_hbm, score_hbm, label_hbm, box_hbm, out_hbm,
                     pos_v, buf_v, rows_v, comb_v, idx2_v, plane_v, gout_v,
                     sh1):
  cid = lax.axis_index("c")
  sid = lax.axis_index("s")
  base = pl.multiple_of(sid * CHUNK, 8)

  def zero_body(i, carry):
    buf_v[pl.ds(pl.multiple_of(i * 16, 16), 16)] = jnp.zeros((16,), jnp.int32)
    return carry

  lax.fori_loop(0, SV, zero_body, 0)
  lane16 = lax.iota(jnp.int32, 16)

  @pl.when(sid < NSCAT)
  def _scatter():
    roff = pl.multiple_of(sid * CROWS_PER_SUB, 8)
    pltpu.sync_copy(pos_hbm.at[cid, pl.ds(roff, CROWS_PER_SUB), :], pos_v)
    for j in range(NV):
      p = pos_v[j // 8, pl.ds((j % 8) * 16, 16)]
      e = base + j * 16 + lane16
      plsc.store_scatter(buf_v, [p], e, mask=p >= 0)

  pltpu.sync_copy(buf_v, sh1.at[sid])
  plsc.subcore_barrier()

  soff = pl.multiple_of(sid * SUBSLOTS, 8)
  for r in range(NSUB):
    pltpu.sync_copy(sh1.at[r, pl.ds(soff, SUBSLOTS)], rows_v.at[r])
  for v in range(SUBSLOTS // 16):
    acc = jnp.zeros((16,), jnp.int32)
    for r in range(NSUB):
      acc = acc + rows_v[r, pl.ds(v * 16, 16)]
    comb_v[pl.ds(v * 16, 16)] = acc
  for v in range(SUBSLOTS // 16, 128 // 16):
    comb_v[pl.ds(v * 16, 16)] = jnp.zeros((16,), jnp.int32)
  plsc.subcore_barrier()
  pltpu.sync_copy(comb_v, sh1.at[sid, pl.ds(0, 128)])
  plsc.subcore_barrier()

  @pl.when(sid < 6)
  def _gather():
    for r in range(NSUB):
      pltpu.sync_copy(sh1.at[r, pl.ds(0, SUBSLOTS)], idx2_v.at[r])
    for ch in range(4):
      @pl.when(sid == ch)
      def _load_box(ch=ch):
        pltpu.sync_copy(box_hbm.at[cid, ch], plane_v)
    @pl.when(sid == 4)
    def _load_score():
      pltpu.sync_copy(score_hbm.at[cid], plane_v)
    @pl.when(sid == 5)
    def _load_label():
      pltpu.sync_copy(label_hbm.at[cid], plane_v)
    for g in range(SV):
      iv = idx2_v[g // 4, pl.ds((g % 4) * 16, 16)]
      gout_v[g // 8, pl.ds((g % 8) * 16, 16)] = plsc.load_gather(
          plane_v, [jnp.right_shift(iv, 7), jnp.bitwise_and(iv, LANES - 1)])
    pltpu.sync_copy(gout_v, out_hbm.at[cid, sid])


@functools.cache
def _sc_compact():
  return pl.kernel(
      _sc_compact_body,
      mesh=plsc.VectorSubcoreMesh(core_axis_name="c", subcore_axis_name="s"),
      compiler_params=pltpu.CompilerParams(needs_layout_passes=False),
      out_type=jax.ShapeDtypeStruct((B, 6, CROWS, LANES), jnp.float32),
      scratch_types=[
          pltpu.VMEM((CROWS_PER_SUB, LANES), jnp.int32),
          pltpu.VMEM((SLOTS,), jnp.int32),
          pltpu.VMEM((NSUB, SUBSLOTS), jnp.int32),
          pltpu.VMEM((128,), jnp.int32),
          pltpu.VMEM((NSUB, SUBSLOTS), jnp.int32),
          pltpu.VMEM((ROWS, LANES), jnp.float32),
          pltpu.VMEM((CROWS, LANES), jnp.float32),
          pltpu.VMEM_SHARED((NSUB, SLOTS), jnp.int32),
      ],
  )


def _nms_kernel(cpt_ref, out_ref):
  x1 = cpt_ref[0, 0]
  y1 = cpt_ref[0, 1]
  x2 = cpt_ref[0, 2]
  y2 = cpt_ref[0, 3]
  scores = cpt_ref[0, 4]
  labels = cpt_ref[0, 5]
  lin = (jax.lax.broadcasted_iota(jnp.int32, (CROWS, LANES), 0) * LANES
         + jax.lax.broadcasted_iota(jnp.int32, (CROWS, LANES), 1))
  valid = lin < PRE_NMS_TOP_N
  s0 = jnp.where(valid & (scores > SCORE_THR), scores, -1e9)
  offs = labels * CLASS_OFFSET
  x1o = x1 + offs
  y1o = y1 + offs
  x2o = x2 + offs
  y2o = y2 + offs
  areas = jnp.maximum(x2o - x1o, 0.0) * jnp.maximum(y2o - y1o, 0.0)
  lane_i = jax.lax.broadcasted_iota(jnp.int32, (1, LANES), 1)

  def body(i, s):
    sc = jnp.max(s)
    il = jnp.min(jnp.where(s == sc, lin, jnp.int32(2**30)))
    r = il // LANES
    c = il % LANES

    def pick(row):
      return jnp.sum(jnp.where(lane_i == c, row, 0.0))

    bx1 = pick(cpt_ref[0, 0, pl.ds(r, 1), :])
    by1 = pick(cpt_ref[0, 1, pl.ds(r, 1), :])
    bx2 = pick(cpt_ref[0, 2, pl.ds(r, 1), :])
    by2 = pick(cpt_ref[0, 3, pl.ds(r, 1), :])
    lab = pick(cpt_ref[0, 5, pl.ds(r, 1), :])
    det = jnp.concatenate([
        jnp.full((1, 1), v, jnp.float32)
        for v in (bx1, by1, bx2, by2, sc, lab)
    ], axis=1)
    det = jnp.where(sc > -1e8, det, jnp.zeros_like(det))
    out_ref[0, pl.ds(i, 1), :] = det
    o = lab * CLASS_OFFSET
    b0 = bx1 + o
    b1 = by1 + o
    b2 = bx2 + o
    b3 = by2 + o
    xx1 = jnp.maximum(b0, x1o)
    yy1 = jnp.maximum(b1, y1o)
    xx2 = jnp.minimum(b2, x2o)
    yy2 = jnp.minimum(b3, y2o)
    inter = jnp.maximum(xx2 - xx1, 0.0) * jnp.maximum(yy2 - yy1, 0.0)
    a = jnp.maximum(b2 - b0, 0.0) * jnp.maximum(b3 - b1, 0.0)
    iou = inter / (a + areas - inter + 1e-9)
    s = jnp.where(iou > IOU_THR, -1e9, s)
    s = jnp.where(lin == il, -1e9, s)
    return s

  jax.lax.fori_loop(0, POST_NMS_TOP_N, body, s0)


@jax.jit
def kernel(cls_pred, loc_pred, ctr_pred, locations):
  scores, labels, boxes = pl.pallas_call(
      _map_kernel,
      grid=(B, KB),
      in_specs=[
          pl.BlockSpec((1, BLK, NUM_CLASSES), lambda b, k: (b, k, 0)),
          pl.BlockSpec((1, BLK, 4), lambda b, k: (b, k, 0)),
          pl.BlockSpec((1, BLK, 1), lambda b, k: (b, k, 0)),
          pl.BlockSpec((BLK, 2), lambda b, k: (k, 0)),
      ],
      out_specs=[
          pl.BlockSpec((1, BLKP // LANES, LANES), lambda b, k: (b, k, 0)),
          pl.BlockSpec((1, BLKP // LANES, LANES), lambda b, k: (b, k, 0)),
          pl.BlockSpec((1, 4, BLKP // LANES, LANES),
                       lambda b, k: (b, 0, k, 0)),
      ],
      out_shape=[
          jax.ShapeDtypeStruct((B, ROWS, LANES), jnp.float32),
          jax.ShapeDtypeStruct((B, ROWS, LANES), jnp.float32),
          jax.ShapeDtypeStruct((B, 4, ROWS, LANES), jnp.float32),
      ],
  )(cls_pred, loc_pred, ctr_pred, locations)

  pos = pl.pallas_call(
      _select_kernel,
      grid=(B,),
      in_specs=[pl.BlockSpec((1, ROWS, LANES), lambda b: (b, 0, 0))],
      out_specs=pl.BlockSpec((1, ROWS, LANES), lambda b: (b, 0, 0)),
      out_shape=jax.ShapeDtypeStruct((B, ROWS, LANES), jnp.int32),
  )(scores)

  compact = _sc_compact()(pos, scores, labels, boxes)

  dets = pl.pallas_call(
      _nms_kernel,
      grid=(B,),
      in_specs=[pl.BlockSpec((1, 6, CROWS, LANES), lambda b: (b, 0, 0, 0))],
      out_specs=pl.BlockSpec((1, POST_NMS_TOP_N, 6), lambda b: (b, 0, 0)),
      out_shape=jax.ShapeDtypeStruct((B, POST_NMS_TOP_N, 6), jnp.float32),
  )(compact)
  return dets
